# baseline (speedup 1.0000x reference)
import jax
import jax.numpy as jnp
from jax import lax
from jax.experimental import pallas as pl
from jax.experimental.pallas import tpu as pltpu
from jax.experimental.pallas import tpu_sc as plsc

BATCH = 16384
B_PER_W = 1024


def _body(uids_hbm, iids_hbm, utab_hbm, itab_hbm, out_hbm, out_v, sem):
    wid = lax.axis_index("s")
    base = wid * B_PER_W

    def step(g, carry):
        out_v[pl.ds(g * 16, 16)] = jnp.full((16,), 1.0, jnp.float32)
        return carry

    lax.fori_loop(0, B_PER_W // 16, step, 0)
    pltpu.sync_copy(out_v, out_hbm.at[pl.ds(base, B_PER_W)])


@jax.jit
def kernel(user_ids, item_ids, user_table, item_table):
    mesh = plsc.VectorSubcoreMesh(core_axis_name="c", subcore_axis_name="s",
                                  num_cores=1)
    run = pl.kernel(
        _body,
        out_type=jax.ShapeDtypeStruct((BATCH,), jnp.float32),
        mesh=mesh,
        compiler_params=pltpu.CompilerParams(needs_layout_passes=False,
                                             skip_device_barrier=True),
        scratch_types=[
            pltpu.VMEM((B_PER_W,), jnp.float32),
            pltpu.SemaphoreType.DMA,
        ],
    )
    return run(user_ids.astype(jnp.int32), item_ids.astype(jnp.int32),
               user_table, item_table)


# speedup vs baseline: 3.7792x; 3.7792x over previous
import jax
import jax.numpy as jnp
from jax import lax
from jax.experimental import pallas as pl
from jax.experimental.pallas import tpu as pltpu
from jax.experimental.pallas import tpu_sc as plsc

LATENT_DIM = 16
BATCH = 16384
NUM_WORKERS = 32
B_PER_W = BATCH // NUM_WORKERS
LANES = 16
CHUNK = 16
N_CHUNKS = B_PER_W // CHUNK
TILE_W = 128


def _lfm_body(uids_hbm, iids_hbm, utab_hbm, itab_hbm, out_hbm,
              uidx_v, iidx_v, ubuf_v, ibuf_v, out_v, usem, isem):
    wid = lax.axis_index("s") * 2 + lax.axis_index("c")
    base = wid * B_PER_W

    pltpu.sync_copy(uids_hbm.at[pl.ds(base, B_PER_W)], uidx_v)
    pltpu.sync_copy(iids_hbm.at[pl.ds(base, B_PER_W)], iidx_v)

    neg128 = jnp.int32(-128)
    lane_iota = lax.iota(jnp.int32, LANES)
    mask127 = jnp.full((LANES,), 127, jnp.int32)

    def chunk_step(k, carry):
        sl = pl.ds(k * CHUNK, CHUNK)
        uvec = uidx_v[sl]
        ivec = iidx_v[sl]
        ubase = jnp.bitwise_and(uvec, neg128)
        ibase = jnp.bitwise_and(ivec, neg128)
        for t in range(CHUNK):
            dst = pl.ds(t * TILE_W, TILE_W)
            pltpu.async_copy(
                utab_hbm.at[:, pl.ds(pl.multiple_of(ubase[t], TILE_W),
                                     TILE_W)],
                ubuf_v.at[:, dst], usem)
            pltpu.async_copy(
                itab_hbm.at[:, pl.ds(pl.multiple_of(ibase[t], TILE_W),
                                     TILE_W)],
                ibuf_v.at[:, dst], isem)

        def drain(m, c2):
            pltpu.make_async_copy(utab_hbm.at[:, pl.ds(0, TILE_W)],
                                  ubuf_v.at[:, pl.ds(0, TILE_W)],
                                  usem).wait()
            pltpu.make_async_copy(itab_hbm.at[:, pl.ds(0, TILE_W)],
                                  ibuf_v.at[:, pl.ds(0, TILE_W)],
                                  isem).wait()
            return c2

        lax.fori_loop(0, CHUNK, drain, 0)

        usub = jnp.bitwise_and(uvec, mask127)
        isub = jnp.bitwise_and(ivec, mask127)
        ucol = lane_iota * TILE_W + usub
        icol = lane_iota * TILE_W + isub
        acc = jnp.zeros((LANES,), jnp.float32)
        for d in range(LATENT_DIM):
            dv = jnp.full((LANES,), d, jnp.int32)
            uu = plsc.load_gather(ubuf_v, [dv, ucol])
            ii = plsc.load_gather(ibuf_v, [dv, icol])
            acc = acc + uu * ii
        out_v[sl] = acc
        return carry

    lax.fori_loop(0, N_CHUNKS, chunk_step, 0)

    pltpu.sync_copy(out_v, out_hbm.at[pl.ds(base, B_PER_W)])


@jax.jit
def kernel(user_ids, item_ids, user_table, item_table):
    mesh = plsc.VectorSubcoreMesh(core_axis_name="c", subcore_axis_name="s")
    run = pl.kernel(
        _lfm_body,
        out_type=jax.ShapeDtypeStruct((BATCH,), jnp.float32),
        mesh=mesh,
        compiler_params=pltpu.CompilerParams(needs_layout_passes=False),
        scratch_types=[
            pltpu.VMEM((B_PER_W,), jnp.int32),
            pltpu.VMEM((B_PER_W,), jnp.int32),
            pltpu.VMEM((LATENT_DIM, CHUNK * TILE_W), jnp.float32),
            pltpu.VMEM((LATENT_DIM, CHUNK * TILE_W), jnp.float32),
            pltpu.VMEM((B_PER_W,), jnp.float32),
            pltpu.SemaphoreType.DMA,
            pltpu.SemaphoreType.DMA,
        ],
    )
    return run(user_ids.astype(jnp.int32), item_ids.astype(jnp.int32),
               user_table.T, item_table.T)
